# baseline (device time: 25180 ns/iter reference)
import jax
import jax.numpy as jnp
from jax import lax
from jax.experimental import pallas as pl
from jax.experimental.pallas import tpu as pltpu

KQ = 8
NY = 3 * KQ
NX = KQ


def kernel(x):
    m_per, n = x.shape
    qsize = m_per // 4
    chunk = qsize // KQ

    def body(x_ref, out_ref, send_y, recv_y, send_x, recv_x):
        my_x = lax.axis_index("x")
        my_y = lax.axis_index("y")
        nbr_y = (my_x, 1 - my_y)
        nbr_x = (1 - my_x, my_y)

        barrier_sem = pltpu.get_barrier_semaphore()
        for nbr in (nbr_y, nbr_x):
            pl.semaphore_signal(
                barrier_sem, inc=1, device_id=nbr,
                device_id_type=pl.DeviceIdType.MESH,
            )

        mine = my_y * m_per
        theirs = (1 - my_y) * m_per
        fwd_q = 3 * my_x
        skip_q = 3 - 3 * my_x

        q_offs = [fwd_q * qsize, 1 * qsize, 2 * qsize]

        out_ref[pl.ds(mine + q_offs[0], chunk), :] = (
            x_ref[pl.ds(q_offs[0], chunk), :].astype(out_ref.dtype)
        )

        pl.semaphore_wait(barrier_sem, 2)

        y_rdmas = []
        for p in range(NY):
            off = q_offs[p // KQ] + (p % KQ) * chunk
            r = pltpu.make_async_remote_copy(
                src_ref=out_ref.at[pl.ds(mine + off, chunk), :],
                dst_ref=out_ref.at[pl.ds(mine + off, chunk), :],
                send_sem=send_y.at[p],
                recv_sem=recv_y.at[p],
                device_id=nbr_y,
                device_id_type=pl.DeviceIdType.MESH,
            )
            r.start()
            y_rdmas.append(r)
            if p + 1 < NY:
                noff = q_offs[(p + 1) // KQ] + ((p + 1) % KQ) * chunk
                out_ref[pl.ds(mine + noff, chunk), :] = (
                    x_ref[pl.ds(noff, chunk), :].astype(out_ref.dtype)
                )

        out_ref[pl.ds(mine + skip_q * qsize, qsize), :] = (
            x_ref[pl.ds(skip_q * qsize, qsize), :].astype(out_ref.dtype)
        )

        x_rdmas = []
        for c in range(NX):
            y_rdmas[c].wait_recv()
            off = theirs + fwd_q * qsize + c * chunk
            r = pltpu.make_async_remote_copy(
                src_ref=out_ref.at[pl.ds(off, chunk), :],
                dst_ref=out_ref.at[pl.ds(off, chunk), :],
                send_sem=send_x.at[c],
                recv_sem=recv_x.at[c],
                device_id=nbr_x,
                device_id_type=pl.DeviceIdType.MESH,
            )
            r.start()
            x_rdmas.append(r)

        for p in range(NX, NY):
            y_rdmas[p].wait_recv()
        for c in range(NX):
            x_rdmas[c].wait_recv()
            x_rdmas[c].wait_send()
        for p in range(NY):
            y_rdmas[p].wait_send()

    return pl.pallas_call(
        body,
        out_shape=jax.ShapeDtypeStruct((2 * m_per, n), jnp.bfloat16),
        in_specs=[pl.BlockSpec(memory_space=pltpu.VMEM)],
        out_specs=pl.BlockSpec(memory_space=pltpu.VMEM),
        scratch_shapes=[
            pltpu.SemaphoreType.DMA((NY,)),
            pltpu.SemaphoreType.DMA((NY,)),
            pltpu.SemaphoreType.DMA((NX,)),
            pltpu.SemaphoreType.DMA((NX,)),
        ],
        compiler_params=pltpu.CompilerParams(collective_id=0),
    )(x)


# device time: 22638 ns/iter; 1.1123x vs baseline; 1.1123x over previous
import jax
import jax.numpy as jnp
from jax import lax
from jax.experimental import pallas as pl
from jax.experimental.pallas import tpu as pltpu

C = 16


def kernel(x):
    m_per, n = x.shape
    half = m_per // 2
    chunk = half // C

    def body(x_ref, out_ref, send_y, recv_y, send_x, recv_x):
        my_x = lax.axis_index("x")
        my_y = lax.axis_index("y")
        nbr_y = (my_x, 1 - my_y)
        nbr_x = (1 - my_x, my_y)

        barrier_sem = pltpu.get_barrier_semaphore()
        for nbr in (nbr_y, nbr_x):
            pl.semaphore_signal(
                barrier_sem, inc=1, device_id=nbr,
                device_id_type=pl.DeviceIdType.MESH,
            )

        send_base = my_y * m_per + my_x * half
        other_base = my_y * m_per + (1 - my_x) * half
        recv_base = (1 - my_y) * m_per + my_x * half

        out_ref[pl.ds(send_base, chunk), :] = (
            x_ref[pl.ds(my_x * half, chunk), :].astype(out_ref.dtype)
        )

        pl.semaphore_wait(barrier_sem, 2)

        y_rdmas = []
        for c in range(C):
            r = pltpu.make_async_remote_copy(
                src_ref=out_ref.at[pl.ds(send_base + c * chunk, chunk), :],
                dst_ref=out_ref.at[pl.ds(send_base + c * chunk, chunk), :],
                send_sem=send_y.at[c],
                recv_sem=recv_y.at[c],
                device_id=nbr_y,
                device_id_type=pl.DeviceIdType.MESH,
            )
            r.start()
            y_rdmas.append(r)
            if c + 1 < C:
                out_ref[pl.ds(send_base + (c + 1) * chunk, chunk), :] = (
                    x_ref[pl.ds(my_x * half + (c + 1) * chunk, chunk), :]
                    .astype(out_ref.dtype)
                )

        out_ref[pl.ds(other_base, half), :] = (
            x_ref[pl.ds((1 - my_x) * half, half), :].astype(out_ref.dtype)
        )

        x_rdmas = []
        for c in range(C):
            y_rdmas[c].wait_recv()
            r = pltpu.make_async_remote_copy(
                src_ref=out_ref.at[pl.ds(recv_base + c * chunk, chunk), :],
                dst_ref=out_ref.at[pl.ds(recv_base + c * chunk, chunk), :],
                send_sem=send_x.at[c],
                recv_sem=recv_x.at[c],
                device_id=nbr_x,
                device_id_type=pl.DeviceIdType.MESH,
            )
            r.start()
            x_rdmas.append(r)

        for c in range(C):
            x_rdmas[c].wait_recv()
            x_rdmas[c].wait_send()
            y_rdmas[c].wait_send()

    return pl.pallas_call(
        body,
        out_shape=jax.ShapeDtypeStruct((2 * m_per, n), jnp.bfloat16),
        in_specs=[pl.BlockSpec(memory_space=pltpu.VMEM)],
        out_specs=pl.BlockSpec(memory_space=pltpu.VMEM),
        scratch_shapes=[
            pltpu.SemaphoreType.DMA((C,)),
            pltpu.SemaphoreType.DMA((C,)),
            pltpu.SemaphoreType.DMA((C,)),
            pltpu.SemaphoreType.DMA((C,)),
        ],
        compiler_params=pltpu.CompilerParams(collective_id=0),
    )(x)
